# baseline (speedup 1.0000x reference)
import jax
import jax.numpy as jnp
from jax import lax
from jax.experimental import pallas as pl
from jax.experimental.pallas import tpu as pltpu

INPUT_DIM = 8192
HIDDEN = 512
K = 256
BATCH = 128

BK = 4096
BN = 4096
NB1 = INPUT_DIM // BK
NB2 = INPUT_DIM // BN
P2 = NB1
P4 = NB1 + NB2
STEPS = NB1 + 2 * NB2

QUAD_ITERS = 6


def _body(clr_ref, w1_ref, b1_ref, w2_ref, b2_ref,
          imp_out_ref, gated_ref, h_ref, thr_ref):
    j = pl.program_id(0)

    @pl.when(j < P2)
    def _():
        @pl.when(j == 0)
        def _():
            h_ref[...] = jnp.zeros_like(h_ref)
        h_ref[...] += jnp.dot(clr_ref[...], w1_ref[...],
                              preferred_element_type=jnp.float32)

    @pl.when(j == P2)
    def _():
        x = h_ref[...] + b1_ref[...]
        h_ref[...] = 0.5 * x * (1.0 + lax.erf(x * 0.7071067811865476))

    @pl.when((j >= P2) & (j < P4))
    def _():
        nb = j - P2
        scores = jnp.dot(h_ref[...], w2_ref[...],
                         preferred_element_type=jnp.float32) + b2_ref[...]
        imp_out_ref[:, pl.ds(nb * BN, BN)] = jax.nn.sigmoid(scores)

    @pl.when(j == P4)
    def _():
        lo0 = jnp.zeros((BATCH, 1), jnp.float32)
        w0 = jnp.ones((BATCH, 1), jnp.float32)

        def it(_, carry):
            lo, w = carry
            q = 0.25 * w
            x = imp_out_ref[...]
            m1 = lo + q
            m2 = lo + 2.0 * q
            m3 = lo + 3.0 * q
            c1 = jnp.sum((x >= m1).astype(jnp.int32), axis=1, keepdims=True)
            c2 = jnp.sum((x >= m2).astype(jnp.int32), axis=1, keepdims=True)
            c3 = jnp.sum((x >= m3).astype(jnp.int32), axis=1, keepdims=True)
            nq = ((c1 >= K).astype(jnp.float32) + (c2 >= K).astype(jnp.float32)
                  + (c3 >= K).astype(jnp.float32))
            return lo + nq * q, q

        lo, _ = lax.fori_loop(0, QUAD_ITERS, it, (lo0, w0))
        thr_ref[...] = jnp.broadcast_to(lo, (BATCH, 128))

    @pl.when(j >= P4)
    def _():
        nb = j - P4
        imp = imp_out_ref[:, pl.ds(nb * BN, BN)]
        thr = thr_ref[:, 0:1]
        gated_ref[:, pl.ds(nb * BN, BN)] = (
            clr_ref[...] * jax.nn.sigmoid(10.0 * (imp - thr)))


def kernel(clr, W1, b1, W2, b2):
    b1r = b1.reshape(1, HIDDEN)
    b2r = b2.reshape(1, INPUT_DIM)

    def clr_map(j):
        return (0, jnp.where(j < P2, jnp.minimum(j, NB1 - 1),
                             jnp.clip(j - P4, 0, NB2 - 1)))

    grid = (STEPS,)
    in_specs = [
        pl.BlockSpec((BATCH, BK), clr_map),
        pl.BlockSpec((BK, HIDDEN), lambda j: (jnp.minimum(j, NB1 - 1), 0)),
        pl.BlockSpec((1, HIDDEN), lambda j: (0, 0)),
        pl.BlockSpec((HIDDEN, BN),
                     lambda j: (0, jnp.clip(j - P2, 0, NB2 - 1))),
        pl.BlockSpec((1, BN), lambda j: (0, jnp.clip(j - P2, 0, NB2 - 1))),
    ]
    out_specs = [
        pl.BlockSpec((BATCH, INPUT_DIM), lambda j: (0, 0)),
        pl.BlockSpec((BATCH, INPUT_DIM), lambda j: (0, 0)),
    ]

    imp, gated = pl.pallas_call(
        _body,
        grid=grid,
        in_specs=in_specs,
        out_specs=out_specs,
        out_shape=[
            jax.ShapeDtypeStruct((BATCH, INPUT_DIM), jnp.float32),
            jax.ShapeDtypeStruct((BATCH, INPUT_DIM), jnp.float32),
        ],
        scratch_shapes=[
            pltpu.VMEM((BATCH, HIDDEN), jnp.float32),
            pltpu.VMEM((BATCH, 128), jnp.float32),
        ],
        compiler_params=pltpu.CompilerParams(
            dimension_semantics=("arbitrary",),
        ),
    )(clr, W1, b1r, W2, b2r)
    return (gated, imp)


# speedup vs baseline: 1.0452x; 1.0452x over previous
import jax
import jax.numpy as jnp
from jax import lax
from jax.experimental import pallas as pl
from jax.experimental.pallas import tpu as pltpu

INPUT_DIM = 8192
HIDDEN = 512
K = 256
BATCH = 128

BK = 4096
BN = 4096
NB1 = INPUT_DIM // BK
NB2 = INPUT_DIM // BN
P2 = NB1
P4 = NB1 + NB2
STEPS = NB1 + 2 * NB2

BISECT_ITERS = 13


def _body(clr_ref, w1_ref, b1_ref, w2_ref, b2_ref,
          imp_out_ref, gated_ref, h_ref, thr_ref):
    j = pl.program_id(0)

    @pl.when(j < P2)
    def _():
        @pl.when(j == 0)
        def _():
            h_ref[...] = jnp.zeros_like(h_ref)
        h_ref[...] += jnp.dot(clr_ref[...], w1_ref[...],
                              preferred_element_type=jnp.float32)

    @pl.when(j == P2)
    def _():
        x = h_ref[...] + b1_ref[...]
        h_ref[...] = 0.5 * x * (1.0 + lax.erf(x * 0.7071067811865476))

    @pl.when((j >= P2) & (j < P4))
    def _():
        nb = j - P2
        scores = jnp.dot(h_ref[...], w2_ref[...],
                         preferred_element_type=jnp.float32) + b2_ref[...]
        imp_out_ref[:, pl.ds(nb * BN, BN)] = jax.nn.sigmoid(scores)

    @pl.when(j == P4)
    def _():
        lo0 = jnp.zeros((BATCH, 1), jnp.float32)
        hi0 = jnp.ones((BATCH, 1), jnp.float32)

        def it(_, carry):
            lo, hi = carry
            mid = 0.5 * (lo + hi)
            cnt = jnp.sum((imp_out_ref[...] >= mid).astype(jnp.int32),
                          axis=1, keepdims=True)
            ge = cnt >= K
            return jnp.where(ge, mid, lo), jnp.where(ge, hi, mid)

        lo, _ = lax.fori_loop(0, BISECT_ITERS, it, (lo0, hi0))
        thr_ref[...] = jnp.broadcast_to(lo, (BATCH, 128))

    @pl.when(j >= P4)
    def _():
        nb = j - P4
        imp = imp_out_ref[:, pl.ds(nb * BN, BN)]
        thr = thr_ref[:, 0:1]
        gated_ref[:, pl.ds(nb * BN, BN)] = (
            clr_ref[...] * jax.nn.sigmoid(10.0 * (imp - thr)))


def kernel(clr, W1, b1, W2, b2):
    b1r = b1.reshape(1, HIDDEN)
    b2r = b2.reshape(1, INPUT_DIM)

    def clr_map(j):
        return (0, jnp.where(j < P2, jnp.minimum(j, NB1 - 1),
                             jnp.clip(j - P4, 0, NB2 - 1)))

    grid = (STEPS,)
    in_specs = [
        pl.BlockSpec((BATCH, BK), clr_map),
        pl.BlockSpec((BK, HIDDEN), lambda j: (jnp.minimum(j, NB1 - 1), 0)),
        pl.BlockSpec((1, HIDDEN), lambda j: (0, 0)),
        pl.BlockSpec((HIDDEN, BN),
                     lambda j: (0, jnp.clip(j - P2, 0, NB2 - 1))),
        pl.BlockSpec((1, BN), lambda j: (0, jnp.clip(j - P2, 0, NB2 - 1))),
    ]
    out_specs = [
        pl.BlockSpec((BATCH, INPUT_DIM), lambda j: (0, 0)),
        pl.BlockSpec((BATCH, INPUT_DIM), lambda j: (0, 0)),
    ]

    imp, gated = pl.pallas_call(
        _body,
        grid=grid,
        in_specs=in_specs,
        out_specs=out_specs,
        out_shape=[
            jax.ShapeDtypeStruct((BATCH, INPUT_DIM), jnp.float32),
            jax.ShapeDtypeStruct((BATCH, INPUT_DIM), jnp.float32),
        ],
        scratch_shapes=[
            pltpu.VMEM((BATCH, HIDDEN), jnp.float32),
            pltpu.VMEM((BATCH, 128), jnp.float32),
        ],
        compiler_params=pltpu.CompilerParams(
            dimension_semantics=("arbitrary",),
        ),
    )(clr, W1, b1r, W2, b2r)
    return (gated, imp)
